# baseline (speedup 1.0000x reference)
import functools

import jax
import jax.numpy as jnp
from jax import lax
from jax.experimental import pallas as pl
from jax.experimental.pallas import tpu as pltpu
from jax.experimental.pallas import tpu_sc as plsc

BATCH = 16384
NF = 100
ED = 32
OD = 2 * ED

_info = plsc.get_sparse_core_info()
NC, NS, L = _info.num_cores, _info.num_subcores, _info.num_lanes
NW = NC * NS
ROWS_PER_W = BATCH // NW
R = 2
NCHUNK = ROWS_PER_W // R
NFP = 112


def _sc_body(x_hbm, m_hbm, bp_hbm, dd_hbm, w_hbm, out_hbm,
             x_v, m_v, bp_v, dd_v, w_v, ob0, ob1, sem_in, sem0, sem1):
    wid = lax.axis_index("s") * NC + lax.axis_index("c")
    base = wid * ROWS_PER_W

    pltpu.sync_copy(bp_hbm, bp_v)
    pltpu.sync_copy(dd_hbm, dd_v)
    pltpu.sync_copy(w_hbm, w_v)

    w_regs = [w_v[pl.ds(k * L, L)] for k in range(OD // L)]

    obs = (ob0, ob1)
    sems = (sem0, sem1)

    def bcast_lane(v16, jl):
        return lax.gather(
            v16, jl[:, None],
            dimension_numbers=lax.GatherDimensionNumbers(
                offset_dims=(), collapsed_slice_dims=(0,),
                start_index_map=(0,)),
            slice_sizes=(1,),
            mode=lax.GatherScatterMode.PROMISE_IN_BOUNDS)

    def do_chunk(c, ob):
        pltpu.sync_copy(x_hbm.at[pl.ds((base + c * R) * NFP, R * NFP)], x_v)
        pltpu.sync_copy(m_hbm.at[pl.ds((base + c * R) * NFP, R * NFP)], m_v)

        def jbody(j, _):
            jq16 = (j // L) * L
            jl = jnp.full((L,), j - jq16, jnp.int32)
            bp_regs = [bp_v[pl.ds(j * OD + k * L, L)] for k in range(OD // L)]
            dd_regs = [dd_v[pl.ds(j * OD + k * L, L)] for k in range(OD // L)]
            for r in range(R):
                x16 = x_v[pl.ds(r * NFP + jq16, L)]
                m16 = m_v[pl.ds(r * NFP + jq16, L)]
                xv = bcast_lane(x16, jl)
                mv = bcast_lane(m16, jl)
                a = xv - xv * mv
                for k in range(OD // L):
                    ob[r, j, pl.ds(k * L, L)] = (
                        a * w_regs[k] + (mv * dd_regs[k] + bp_regs[k]))
            return _

        lax.fori_loop(0, NF, jbody, None)

    def c2body(c2, _):
        for p in range(2):
            c = 2 * c2 + p

            @pl.when(c2 > 0)
            def _wait():
                pltpu.make_async_copy(
                    obs[p], out_hbm.at[pl.ds(0, R)], sems[p]).wait()

            do_chunk(c, obs[p])
            pltpu.async_copy(
                obs[p], out_hbm.at[pl.ds(base + c * R, R)], sems[p])
        return _

    lax.fori_loop(0, NCHUNK // 2, c2body, None)
    for p in range(2):
        pltpu.make_async_copy(obs[p], out_hbm.at[pl.ds(0, R)], sems[p]).wait()


_mesh = plsc.VectorSubcoreMesh(core_axis_name="c", subcore_axis_name="s")

_sc_kernel = functools.partial(
    pl.kernel,
    mesh=_mesh,
    out_type=jax.ShapeDtypeStruct((BATCH, NF, OD), jnp.float32),
    scratch_types=[
        pltpu.VMEM((R * NFP,), jnp.float32),
        pltpu.VMEM((R * NFP,), jnp.float32),
        pltpu.VMEM((NF * OD,), jnp.float32),
        pltpu.VMEM((NF * OD,), jnp.float32),
        pltpu.VMEM((OD,), jnp.float32),
        pltpu.VMEM((R, NF, OD), jnp.float32),
        pltpu.VMEM((R, NF, OD), jnp.float32),
        pltpu.SemaphoreType.DMA,
        pltpu.SemaphoreType.DMA,
        pltpu.SemaphoreType.DMA,
    ],
)(_sc_body)


def kernel(x_hat, mask, Wv, bv, missing_table, present_table):
    w = Wv[:, 0]
    w64 = jnp.concatenate([w, jnp.zeros((ED,), jnp.float32)])
    bp = jnp.concatenate(
        [jnp.broadcast_to(bv, (NF, ED)), present_table], axis=1)
    mm = jnp.concatenate(
        [jnp.zeros((NF, ED), jnp.float32), missing_table], axis=1)
    dd = mm - bp
    x_pad = jnp.pad(x_hat, ((0, 0), (0, NFP - NF))).reshape(-1)
    m_pad = jnp.pad(mask, ((0, 0), (0, NFP - NF))).reshape(-1)
    return _sc_kernel(x_pad, m_pad, bp.reshape(-1), dd.reshape(-1), w64)


# speedup vs baseline: 9.1963x; 9.1963x over previous
import jax
import jax.numpy as jnp
from jax.experimental import pallas as pl
from jax.experimental.pallas import tpu as pltpu

BATCH = 16384
NF = 100
ED = 32
OD = 2 * ED
BB = 256


def _body(x_ref, m_ref, w_ref, bp_ref, dd_ref, out_ref):
    x = x_ref[...]
    m = m_ref[...]
    a = x - x * m
    w = w_ref[...]
    bp = bp_ref[...]
    dd = dd_ref[...]
    out_ref[...] = a[:, None, :] * w + (m[:, None, :] * dd + bp)


def kernel(x_hat, mask, Wv, bv, missing_table, present_table):
    w = Wv[:, 0]
    w64 = jnp.concatenate([w, jnp.zeros((ED,), jnp.float32)])
    bp = jnp.concatenate(
        [jnp.broadcast_to(bv, (NF, ED)), present_table], axis=1)
    mm = jnp.concatenate(
        [jnp.zeros((NF, ED), jnp.float32), missing_table], axis=1)
    dd = mm - bp

    xT = x_hat.T
    mT = mask.T

    grid = (BATCH // BB,)
    out_t = pl.pallas_call(
        _body,
        grid=grid,
        in_specs=[
            pl.BlockSpec((NF, BB), lambda i: (0, i)),
            pl.BlockSpec((NF, BB), lambda i: (0, i)),
            pl.BlockSpec((1, OD, 1), lambda i: (0, 0, 0)),
            pl.BlockSpec((NF, OD, 1), lambda i: (0, 0, 0)),
            pl.BlockSpec((NF, OD, 1), lambda i: (0, 0, 0)),
        ],
        out_specs=pl.BlockSpec((NF, OD, BB), lambda i: (0, 0, i)),
        out_shape=jax.ShapeDtypeStruct((NF, OD, BATCH), jnp.float32),
        compiler_params=pltpu.CompilerParams(
            dimension_semantics=("arbitrary",),
        ),
    )(xT, mT, w64.reshape(1, OD, 1), bp[:, :, None], dd[:, :, None])
    return jnp.transpose(out_t, (2, 0, 1))


# speedup vs baseline: 10.6019x; 1.1528x over previous
import jax
import jax.numpy as jnp
from jax.experimental import pallas as pl
from jax.experimental.pallas import tpu as pltpu

BATCH = 16384
NF = 100
ED = 32
OD = 2 * ED
BB = 512


def _body(x_ref, m_ref, w_ref, bp_ref, dd_ref, out_ref):
    x = x_ref[...]
    m = m_ref[...]
    a = x - x * m
    w = w_ref[...]
    bp = bp_ref[...]
    dd = dd_ref[...]
    out_ref[...] = a[:, None, :] * w + (m[:, None, :] * dd + bp)


def kernel(x_hat, mask, Wv, bv, missing_table, present_table):
    w = Wv[:, 0]
    w64 = jnp.concatenate([w, jnp.zeros((ED,), jnp.float32)])
    bp = jnp.concatenate(
        [jnp.broadcast_to(bv, (NF, ED)), present_table], axis=1)
    mm = jnp.concatenate(
        [jnp.zeros((NF, ED), jnp.float32), missing_table], axis=1)
    dd = mm - bp

    xT = x_hat.T
    mT = mask.T

    grid = (BATCH // BB,)
    out_t = pl.pallas_call(
        _body,
        grid=grid,
        in_specs=[
            pl.BlockSpec((NF, BB), lambda i: (0, i)),
            pl.BlockSpec((NF, BB), lambda i: (0, i)),
            pl.BlockSpec((1, OD, 1), lambda i: (0, 0, 0)),
            pl.BlockSpec((NF, OD, 1), lambda i: (0, 0, 0)),
            pl.BlockSpec((NF, OD, 1), lambda i: (0, 0, 0)),
        ],
        out_specs=pl.BlockSpec((NF, OD, BB), lambda i: (0, 0, i)),
        out_shape=jax.ShapeDtypeStruct((NF, OD, BATCH), jnp.float32),
        compiler_params=pltpu.CompilerParams(
            dimension_semantics=("arbitrary",),
        ),
    )(xT, mT, w64.reshape(1, OD, 1), bp[:, :, None], dd[:, :, None])
    return jnp.transpose(out_t, (2, 0, 1))
